# Initial kernel scaffold; baseline (speedup 1.0000x reference)
import jax
import jax.numpy as jnp
from jax.experimental import pallas as pl


def kernel(level0_states, level1_states, level2_context, event_mask, obj_mask, W_l2, W_gate, b_gate):
    raise NotImplementedError("write your pallas kernel here")


# speedup vs baseline: 3.2765x; 3.2765x over previous
import functools

import jax
import jax.numpy as jnp
from jax import lax
from jax.experimental import pallas as pl
from jax.experimental.pallas import tpu as pltpu
from jax.experimental.pallas import tpu_sc as plsc

_B, _T, _N, _K = 8, 512, 32, 64
_D = 128
_NC, _NS, _L = 2, 16, 16

@functools.cache
def _rank_sc_kernel():
    mesh = plsc.VectorSubcoreMesh(
        core_axis_name="c", subcore_axis_name="s",
        num_cores=_NC, num_subcores=_NS,
    )

    @functools.partial(
        pl.kernel,
        out_type=jax.ShapeDtypeStruct((_B, _T), jnp.int32),
        mesh=mesh,
        scratch_types=[
            pltpu.VMEM((1, _T), jnp.float32),
            pltpu.VMEM((1, _T), jnp.int32),
        ],
    )
    def _rank_sc(ev_hbm, out_hbm, fbuf, ibuf):
        wid = lax.axis_index("s") * _NC + lax.axis_index("c")

        @pl.when(wid < _B)
        def _():
            pltpu.sync_copy(ev_hbm.at[pl.ds(wid, 1)], fbuf)
            one = jnp.full((_L,), 1, jnp.int32)
            zero = jnp.full((_L,), 0, jnp.int32)
            minus1 = jnp.full((_L,), -1, jnp.int32)
            kvec = jnp.full((_L,), _K, jnp.int32)
            lane = lax.broadcasted_iota(jnp.int32, (_L,), 0)
            last = jnp.full((_L,), _L - 1, jnp.int32)

            def step(i, basev):
                x = fbuf[0, pl.ds(i * _L, _L)]
                ev = jnp.where(x > 0.5, one, zero)
                s = ev
                for sh in (1, 2, 4, 8):
                    g = jnp.take(s, jnp.maximum(lane - sh, 0))
                    s = s + jnp.where(lane >= sh, g, zero)
                excl = s - ev + basev
                r = jnp.where(excl < kvec, excl, minus1)
                ibuf[0, pl.ds(i * _L, _L)] = jnp.where(x > 0.5, r, minus1)
                return basev + jnp.take(s, last)

            lax.fori_loop(0, _T // _L, step, zero)
            pltpu.sync_copy(ibuf, out_hbm.at[pl.ds(wid, 1)])

    return _rank_sc


def _gate_tc(l0_ref, l1_ref, rank_ref, l2_ref, obj_ref, wl2_ref, wg_ref,
             bg_ref, out_ref):
    b = pl.program_id(0)
    rank = rank_ref[0]
    sel = rank >= 0
    cols = lax.broadcasted_iota(jnp.int32, (_T, _K), 1)
    onehot = jnp.where(rank == cols, 1.0, 0.0)
    sel_d = jnp.broadcast_to(sel, (_T, _D))

    dn = (((1,), (1,)), ((), ()))
    l2p = jnp.mean(l2_ref[0], axis=0, keepdims=True)
    l2p = lax.dot_general(l2p, wl2_ref[...], dn,
                          preferred_element_type=jnp.float32)
    wg = wg_ref[...]
    wgc = wg[:, :_D]
    wgl = wg[:, _D:]
    g2 = lax.dot_general(l2p, wgl, dn,
                         preferred_element_type=jnp.float32) + bg_ref[...]

    for n in range(_N):
        x = l0_ref[0, :, n, :]
        v = l1_ref[0, :, n, :]
        g = lax.dot_general(onehot, v, (((1,), (0,)), ((), ())),
                            preferred_element_type=jnp.float32)
        cmb = jnp.where(sel_d, g, x)
        z = lax.dot_general(cmb, wgc, dn,
                            preferred_element_type=jnp.float32) + g2
        gate = jax.nn.sigmoid(z)
        o = gate * l2p + (1.0 - gate) * cmb
        out_ref[0, :, n, :] = o * obj_ref[b, n]


def kernel(level0_states, level1_states, level2_context, event_mask, obj_mask,
           W_l2, W_gate, b_gate):
    n_sum, d_l2 = level2_context.shape[1], level2_context.shape[2]
    rank = _rank_sc_kernel()(event_mask).reshape(_B, _T, 1)
    return pl.pallas_call(
        _gate_tc,
        grid=(_B,),
        in_specs=[
            pl.BlockSpec((1, _T, _N, _D), lambda b: (b, 0, 0, 0)),
            pl.BlockSpec((1, _K, _N, _D), lambda b: (b, 0, 0, 0)),
            pl.BlockSpec((1, _T, 1), lambda b: (b, 0, 0)),
            pl.BlockSpec((1, n_sum, d_l2), lambda b: (b, 0, 0)),
            pl.BlockSpec(memory_space=pltpu.SMEM),
            pl.BlockSpec((_D, d_l2), lambda b: (0, 0)),
            pl.BlockSpec((_D, 2 * _D), lambda b: (0, 0)),
            pl.BlockSpec((1, _D), lambda b: (0, 0)),
        ],
        out_specs=pl.BlockSpec((1, _T, _N, _D), lambda b: (b, 0, 0, 0)),
        out_shape=jax.ShapeDtypeStruct((_B, _T, _N, _D), jnp.float32),
    )(level0_states, level1_states, rank, level2_context, obj_mask, W_l2,
      W_gate, b_gate.reshape(1, _D))
